# baseline (speedup 1.0000x reference)
import functools

import jax
import jax.numpy as jnp
from jax.experimental import pallas as pl
from jax.experimental.pallas import tpu as pltpu

_LANE = 128
_SUBLANE = 8


def _round_up(x, m):
    return (x + m - 1) // m * m


def _ffn_body(x_ref, w1_ref, b1_ref, w2_ref, b2_ref, w3_ref, b3_ref, o_ref):
    xb = x_ref[...].astype(jnp.bfloat16)
    h1 = jnp.dot(xb, w1_ref[...], preferred_element_type=jnp.float32)
    h1 = jnp.maximum(h1 + b1_ref[...], 0.0).astype(jnp.bfloat16)
    h2 = jnp.dot(h1, w2_ref[...], preferred_element_type=jnp.float32)
    h2 = jnp.maximum(h2 + b2_ref[...], 0.0).astype(jnp.bfloat16)
    logits = jnp.dot(h2, w3_ref[...], preferred_element_type=jnp.float32) + b3_ref[...]
    m = jnp.max(logits, axis=-1, keepdims=True)
    e = jnp.exp(logits - m)
    denom = jnp.sum(e, axis=-1, keepdims=True)
    p = e * pl.reciprocal(denom, approx=False)
    o_ref[...] = p[:, : o_ref.shape[1]].astype(o_ref.dtype)


@functools.partial(jax.jit, static_argnames=("tile_m",))
def _feed_forward(x, w1, b1, w2, b2, w3, b3, *, tile_m=512):
    B, in_size = x.shape
    hid = w1.shape[1]
    out_size = w3.shape[1]

    in_p = _round_up(in_size, _LANE)
    hid_p = _round_up(hid, _LANE)
    out_p = _round_up(out_size, _LANE)

    xp = jnp.pad(x, ((0, 0), (0, in_p - in_size))) if in_p != in_size else x
    w1b = jnp.pad(w1, ((0, in_p - in_size), (0, hid_p - hid))).astype(jnp.bfloat16)
    w2b = jnp.pad(w2, ((0, hid_p - hid), (0, hid_p - hid))).astype(jnp.bfloat16)
    w3b = jnp.pad(w3, ((0, hid_p - hid), (0, out_p - out_size))).astype(jnp.bfloat16)
    b1p = jnp.pad(b1, (0, hid_p - hid)).reshape(1, hid_p)
    b2p = jnp.pad(b2, (0, hid_p - hid)).reshape(1, hid_p)
    neg = jnp.finfo(jnp.float32).min
    b3p = jnp.pad(b3, (0, out_p - out_size), constant_values=neg).reshape(1, out_p)

    b_pad8 = _round_up(B, _SUBLANE)
    tm = min(tile_m, b_pad8)
    tm = _round_up(tm, _SUBLANE)
    B_pad = _round_up(b_pad8, tm)
    if B_pad != B:
        xp = jnp.pad(xp, ((0, B_pad - B), (0, 0)))

    grid = (B_pad // tm,)

    flops = 2 * B_pad * (in_p * hid_p + hid_p * hid_p + hid_p * out_p)
    transcendentals = B_pad * out_p
    bytes_accessed = (4 * (B_pad * in_p + B_pad * out_p + 2 * hid_p + out_p)
                      + 2 * (in_p * hid_p + hid_p * hid_p + hid_p * out_p))
    cost = pl.CostEstimate(flops=flops,
                           transcendentals=transcendentals,
                           bytes_accessed=bytes_accessed)

    vmem_est = (4 * (2 * tm * in_p + 2 * tm * out_p + 2 * tm * hid_p
                     + 2 * hid_p + out_p)
                + 2 * 2 * (in_p * hid_p + hid_p * hid_p + hid_p * out_p))
    vmem_limit = int(min(100 * 2**20, max(4 * 2**20, 2 * vmem_est)))

    out = pl.pallas_call(
        _ffn_body,
        out_shape=jax.ShapeDtypeStruct((B_pad, out_size), jnp.float32),
        grid_spec=pltpu.PrefetchScalarGridSpec(
            num_scalar_prefetch=0,
            grid=grid,
            in_specs=[
                pl.BlockSpec((tm, in_p), lambda i: (i, 0)),
                pl.BlockSpec((in_p, hid_p), lambda i: (0, 0)),
                pl.BlockSpec((1, hid_p), lambda i: (0, 0)),
                pl.BlockSpec((hid_p, hid_p), lambda i: (0, 0)),
                pl.BlockSpec((1, hid_p), lambda i: (0, 0)),
                pl.BlockSpec((hid_p, out_p), lambda i: (0, 0)),
                pl.BlockSpec((1, out_p), lambda i: (0, 0)),
            ],
            out_specs=pl.BlockSpec((tm, out_size), lambda i: (i, 0)),
        ),
        compiler_params=pltpu.CompilerParams(
            dimension_semantics=("arbitrary",),
            vmem_limit_bytes=vmem_limit,
        ),
        cost_estimate=cost,
    )(xp, w1b, b1p, w2b, b2p, w3b, b3p)

    return out if B_pad == B else out[:B]


def kernel(x, w1, b1, w2, b2, w3, b3):
    return _feed_forward(x, w1, b1, w2, b2, w3, b3, tile_m=512)


# speedup vs baseline: 1.4660x; 1.4660x over previous
import functools

import jax
import jax.numpy as jnp
from jax.experimental import pallas as pl
from jax.experimental.pallas import tpu as pltpu

_SUBLANE = 8


def _round_up(x, m):
    return (x + m - 1) // m * m


def _ffn_body(x_ref, w1_ref, b1_ref, w2_ref, b2_ref, w3t_ref, b3_ref, o_ref,
              w1s, w2s, w3s):
    @pl.when(pl.program_id(0) == 0)
    def _stage_weights():
        w1s[...] = w1_ref[...].astype(jnp.bfloat16)
        w2s[...] = w2_ref[...].astype(jnp.bfloat16)
        w3s[...] = w3t_ref[...].astype(jnp.bfloat16)

    xb = x_ref[...].astype(jnp.bfloat16)
    h1 = jnp.dot(xb, w1s[...], preferred_element_type=jnp.float32)
    h1 = jnp.maximum(h1 + b1_ref[...], 0.0).astype(jnp.bfloat16)
    h2 = jnp.dot(h1, w2s[...], preferred_element_type=jnp.float32)
    h2 = jnp.maximum(h2 + b2_ref[...], 0.0).astype(jnp.bfloat16)
    logits_t = jax.lax.dot_general(
        w3s[...], h2,
        dimension_numbers=(((1,), (1,)), ((), ())),
        preferred_element_type=jnp.float32,
    ) + b3_ref[...]
    m = jnp.max(logits_t, axis=0, keepdims=True)
    e = jnp.exp(logits_t - m)
    denom = jnp.sum(e, axis=0, keepdims=True)
    o_ref[...] = e * pl.reciprocal(denom, approx=False)


@functools.partial(jax.jit, static_argnames=("tile_m",))
def _feed_forward(x, w1, b1, w2, b2, w3, b3, *, tile_m=512):
    B, in_size = x.shape
    hid = w1.shape[1]
    out_size = w3.shape[1]

    b1r = b1.reshape(1, hid)
    b2r = b2.reshape(1, hid)
    b3r = b3.reshape(out_size, 1)
    w3t = w3.T

    tm = min(tile_m, _round_up(B, _SUBLANE))
    tm = _round_up(tm, _SUBLANE)
    B_pad = _round_up(B, tm)
    xp = x if B_pad == B else jnp.pad(x, ((0, B_pad - B), (0, 0)))

    grid = (B_pad // tm,)

    flops = 2 * B_pad * (in_size * hid + hid * hid + hid * out_size)
    transcendentals = B_pad * out_size
    bytes_accessed = 4 * (B_pad * in_size + B_pad * out_size
                          + in_size * hid + hid * hid + hid * out_size
                          + 2 * hid + out_size)
    cost = pl.CostEstimate(flops=flops,
                           transcendentals=transcendentals,
                           bytes_accessed=bytes_accessed)

    out_t = pl.pallas_call(
        _ffn_body,
        out_shape=jax.ShapeDtypeStruct((out_size, B_pad), jnp.float32),
        grid_spec=pltpu.PrefetchScalarGridSpec(
            num_scalar_prefetch=0,
            grid=grid,
            in_specs=[
                pl.BlockSpec((tm, in_size), lambda i: (i, 0)),
                pl.BlockSpec((in_size, hid), lambda i: (0, 0)),
                pl.BlockSpec((1, hid), lambda i: (0, 0)),
                pl.BlockSpec((hid, hid), lambda i: (0, 0)),
                pl.BlockSpec((1, hid), lambda i: (0, 0)),
                pl.BlockSpec((out_size, hid), lambda i: (0, 0)),
                pl.BlockSpec((out_size, 1), lambda i: (0, 0)),
            ],
            out_specs=pl.BlockSpec((out_size, tm), lambda i: (0, i)),
            scratch_shapes=[
                pltpu.VMEM((in_size, hid), jnp.bfloat16),
                pltpu.VMEM((hid, hid), jnp.bfloat16),
                pltpu.VMEM((out_size, hid), jnp.bfloat16),
            ],
        ),
        compiler_params=pltpu.CompilerParams(
            dimension_semantics=("arbitrary",),
            vmem_limit_bytes=56 * 2**20,
        ),
        cost_estimate=cost,
    )(xp, w1, b1r, w2, b2r, w3t, b3r)

    out = out_t.T
    return out if B_pad == B else out[:B]


def kernel(x, w1, b1, w2, b2, w3, b3):
    return _feed_forward(x, w1, b1, w2, b2, w3, b3, tile_m=512)


# speedup vs baseline: 1.4860x; 1.0137x over previous
import functools

import jax
import jax.numpy as jnp
from jax.experimental import pallas as pl
from jax.experimental.pallas import tpu as pltpu

_SUBLANE = 8


def _round_up(x, m):
    return (x + m - 1) // m * m


def _ffn_body(x_ref, w1_ref, b1_ref, w2_ref, b2_ref, w3t_ref, b3_ref, o_ref,
              w1s, w2s, w3s):
    @pl.when(pl.program_id(0) == 0)
    def _stage_weights():
        w1s[...] = w1_ref[...].astype(jnp.bfloat16)
        w2s[...] = w2_ref[...].astype(jnp.bfloat16)
        w3s[...] = w3t_ref[...].astype(jnp.bfloat16)

    xb = x_ref[...].astype(jnp.bfloat16)
    h1 = jnp.dot(xb, w1s[...], preferred_element_type=jnp.float32)
    h1 = jnp.maximum(h1 + b1_ref[...], 0.0).astype(jnp.bfloat16)
    h2 = jnp.dot(h1, w2s[...], preferred_element_type=jnp.float32)
    h2 = jnp.maximum(h2 + b2_ref[...], 0.0).astype(jnp.bfloat16)
    logits_t = jax.lax.dot_general(
        w3s[...], h2,
        dimension_numbers=(((1,), (1,)), ((), ())),
        preferred_element_type=jnp.float32,
    ) + b3_ref[...]
    m = jnp.max(logits_t, axis=0, keepdims=True)
    e = jnp.exp(logits_t - m)
    denom = jnp.sum(e, axis=0, keepdims=True)
    o_ref[...] = e * pl.reciprocal(denom, approx=False)


@functools.partial(jax.jit, static_argnames=("tile_m",))
def _feed_forward(x, w1, b1, w2, b2, w3, b3, *, tile_m=512):
    B, in_size = x.shape
    hid = w1.shape[1]
    out_size = w3.shape[1]

    b1r = b1.reshape(1, hid)
    b2r = b2.reshape(1, hid)
    b3r = b3.reshape(out_size, 1)
    w3t = w3.T

    tm = min(tile_m, _round_up(B, _SUBLANE))
    tm = _round_up(tm, _SUBLANE)
    B_pad = _round_up(B, tm)
    xp = x if B_pad == B else jnp.pad(x, ((0, B_pad - B), (0, 0)))

    grid = (B_pad // tm,)

    flops = 2 * B_pad * (in_size * hid + hid * hid + hid * out_size)
    transcendentals = B_pad * out_size
    bytes_accessed = 4 * (B_pad * in_size + B_pad * out_size
                          + in_size * hid + hid * hid + hid * out_size
                          + 2 * hid + out_size)
    cost = pl.CostEstimate(flops=flops,
                           transcendentals=transcendentals,
                           bytes_accessed=bytes_accessed)

    out_t = pl.pallas_call(
        _ffn_body,
        out_shape=jax.ShapeDtypeStruct((out_size, B_pad), jnp.float32),
        grid_spec=pltpu.PrefetchScalarGridSpec(
            num_scalar_prefetch=0,
            grid=grid,
            in_specs=[
                pl.BlockSpec((tm, in_size), lambda i: (i, 0)),
                pl.BlockSpec((in_size, hid), lambda i: (0, 0)),
                pl.BlockSpec((1, hid), lambda i: (0, 0)),
                pl.BlockSpec((hid, hid), lambda i: (0, 0)),
                pl.BlockSpec((1, hid), lambda i: (0, 0)),
                pl.BlockSpec((out_size, hid), lambda i: (0, 0)),
                pl.BlockSpec((out_size, 1), lambda i: (0, 0)),
            ],
            out_specs=pl.BlockSpec((out_size, tm), lambda i: (0, i)),
            scratch_shapes=[
                pltpu.VMEM((in_size, hid), jnp.bfloat16),
                pltpu.VMEM((hid, hid), jnp.bfloat16),
                pltpu.VMEM((out_size, hid), jnp.bfloat16),
            ],
        ),
        compiler_params=pltpu.CompilerParams(
            dimension_semantics=("arbitrary",),
            vmem_limit_bytes=56 * 2**20,
        ),
        cost_estimate=cost,
    )(xp, w1, b1r, w2, b2r, w3t, b3r)

    out = out_t.T
    return out if B_pad == B else out[:B]


def kernel(x, w1, b1, w2, b2, w3, b3):
    return _feed_forward(x, w1, b1, w2, b2, w3, b3, tile_m=1024)
